# Initial kernel scaffold; baseline (speedup 1.0000x reference)
import jax
import jax.numpy as jnp
from jax.experimental import pallas as pl


def kernel(x, edge_index, W0, b0, W1, b1, W2, b2):
    raise NotImplementedError("write your pallas kernel here")


# speedup vs baseline: 160.1092x; 160.1092x over previous
import functools

import jax
import jax.numpy as jnp
from jax import lax
from jax.experimental import pallas as pl
from jax.experimental.pallas import tpu as pltpu
from jax.experimental.pallas import tpu_sc as plsc

N = 10000
NPAD = 10240
E = 320000
D = 128
ALPHA = 0.2
NC = 2
NS = 16
NW = NC * NS
EPAD = NW * 10240
EW = EPAD // NW
SCHUNK = 128
NK = EW // SCHUNK
CN = NPAD // NS
FW = NPAD // NW

_mesh = plsc.VectorSubcoreMesh(core_axis_name="c", subcore_axis_name="s")
_sc_params = pltpu.CompilerParams(needs_layout_passes=False)

_f32 = jnp.float32
_sds = jax.ShapeDtypeStruct


def _worker_ids():
    cid = lax.axis_index("c")
    sid = lax.axis_index("s")
    return cid, sid, sid * NC + cid


def _fill(ref, n_vecs, value):
    v = jnp.full((16,), value, _f32)

    @plsc.parallel_loop(0, n_vecs, unroll=4)
    def _(i):
        ref[pl.ds(i * 16, 16)] = v


def _stage(pairs, sem):
    descs = [pltpu.async_copy(s, d, sem) for s, d in pairs]
    for dsc in descs:
        dsc.wait()


def _scatter_chunks(src_val_ref, dst2_v, agg_sp, sem):
    descs = []
    for k in range(NK):
        descs.append(
            pltpu.async_copy(
                src_val_ref.at[pl.ds(k * SCHUNK, SCHUNK)],
                agg_sp.at[dst2_v.at[k]],
                sem,
                add=True,
            )
        )
    for dsc in descs:
        dsc.wait()


def _zero_and_barrier(zero_v, agg_sp, sid):
    _fill(zero_v, CN // 16, 0.0)
    pltpu.sync_copy(zero_v, agg_sp.at[pl.ds(sid * CN, CN)])
    plsc.subcore_barrier()


def _readback(agg_sp, out_v, outA, outB, cid, sid):
    plsc.subcore_barrier()
    chunk = pl.ds(sid * CN, CN)
    pltpu.sync_copy(agg_sp.at[chunk], out_v)

    @pl.when(cid == 0)
    def _():
        pltpu.sync_copy(out_v, outA.at[chunk])

    @pl.when(cid == 1)
    def _():
        pltpu.sync_copy(out_v, outB.at[chunk])


def _rsqrt16(x):
    i = lax.bitcast_convert_type(x, jnp.int32)
    i = jnp.int32(0x5F3759DF) - (i >> 1)
    y = lax.bitcast_convert_type(i, _f32)
    y = y * (1.5 - 0.5 * x * y * y)
    y = y * (1.5 - 0.5 * x * y * y)
    y = y * (1.5 - 0.5 * x * y * y)
    return y


@functools.partial(
    pl.kernel,
    out_type=(_sds((NPAD,), _f32), _sds((NPAD,), _f32)),
    mesh=_mesh,
    compiler_params=_sc_params,
    scratch_types=[
        pltpu.VMEM((NK, SCHUNK), jnp.int32),
        pltpu.VMEM((SCHUNK,), _f32),
        pltpu.VMEM((CN,), _f32),
        pltpu.VMEM((CN,), _f32),
        pltpu.VMEM_SHARED((NPAD,), _f32),
        pltpu.SemaphoreType.DMA,
    ],
)
def _deg_kernel(dst3, degA, degB, dst2_v, ones_v, zero_v, out_v, agg_sp, sem):
    cid, sid, wid = _worker_ids()
    pltpu.sync_copy(dst3.at[wid], dst2_v)
    _fill(ones_v, SCHUNK // 16, 1.0)
    _zero_and_barrier(zero_v, agg_sp, sid)
    descs = [
        pltpu.async_copy(ones_v, agg_sp.at[dst2_v.at[k]], sem, add=True)
        for k in range(NK)
    ]
    for dsc in descs:
        dsc.wait()
    _readback(agg_sp, out_v, degA, degB, cid, sid)


def _h0_body(x_ref, w_ref, b_ref, o_ref):
    o_ref[...] = (
        jnp.dot(x_ref[...], w_ref[...], preferred_element_type=_f32) + b_ref[0]
    )


def _h0_tc(x, W0, b0):
    blk = 1000
    return pl.pallas_call(
        _h0_body,
        grid=(N // blk,),
        in_specs=[
            pl.BlockSpec((blk, D), lambda i: (i, 0)),
            pl.BlockSpec((D, 1), lambda i: (0, 0)),
            pl.BlockSpec(memory_space=pltpu.SMEM),
        ],
        out_specs=pl.BlockSpec((blk, 1), lambda i: (i, 0)),
        out_shape=_sds((N, 1), _f32),
    )(x, W0, b0)


def _gather_msgs(src_v, u_v, msg_v):
    @plsc.parallel_loop(0, EW // 16, unroll=8)
    def _(i):
        o = pl.ds(i * 16, 16)
        msg_v[o] = plsc.load_gather(u_v, [src_v[o]])


@functools.partial(
    pl.kernel,
    out_type=(_sds((NPAD,), _f32), _sds((NPAD,), _f32), _sds((NPAD,), _f32)),
    mesh=_mesh,
    compiler_params=_sc_params,
    scratch_types=[
        pltpu.VMEM((EW,), jnp.int32),
        pltpu.VMEM((NK, SCHUNK), jnp.int32),
        pltpu.VMEM((EW,), _f32),
        pltpu.VMEM((NPAD,), _f32),
        pltpu.VMEM((NPAD,), _f32),
        pltpu.VMEM((NPAD,), _f32),
        pltpu.VMEM((NPAD,), _f32),
        pltpu.VMEM((CN,), _f32),
        pltpu.VMEM((CN,), _f32),
        pltpu.VMEM_SHARED((NPAD,), _f32),
        pltpu.SemaphoreType.DMA,
    ],
)
def _round1_kernel(src2, dst3, degA, degB, h0p, SA, SB, dout,
                   src_v, dst2_v, msg_v, da_v, db_v, h_v, u_v,
                   zero_v, out_v, agg_sp, sem):
    cid, sid, wid = _worker_ids()
    _stage(
        [
            (src2.at[wid], src_v),
            (dst3.at[wid], dst2_v),
            (degA, da_v),
            (degB, db_v),
            (h0p, h_v),
        ],
        sem,
    )
    _zero_and_barrier(zero_v, agg_sp, sid)

    @plsc.parallel_loop(0, NPAD // 16, unroll=8)
    def _(i):
        o = pl.ds(i * 16, 16)
        deg = da_v[o] + db_v[o] + 1.0
        dd = _rsqrt16(deg)
        da_v[o] = dd
        u_v[o] = dd * h_v[o]

    chunk = pl.ds(sid * CN, CN)

    @pl.when(cid == 0)
    def _():
        pltpu.sync_copy(da_v.at[chunk], dout.at[chunk])

    _gather_msgs(src_v, u_v, msg_v)
    _scatter_chunks(msg_v, dst2_v, agg_sp, sem)
    _readback(agg_sp, out_v, SA, SB, cid, sid)


def _make_round():
    @functools.partial(
        pl.kernel,
        out_type=(_sds((NPAD,), _f32), _sds((NPAD,), _f32), _sds((NPAD,), _f32)),
        mesh=_mesh,
    compiler_params=_sc_params,
        scratch_types=[
            pltpu.VMEM((EW,), jnp.int32),
            pltpu.VMEM((NK, SCHUNK), jnp.int32),
            pltpu.VMEM((EW,), _f32),
            pltpu.VMEM((NPAD,), _f32),
            pltpu.VMEM((NPAD,), _f32),
            pltpu.VMEM((NPAD,), _f32),
            pltpu.VMEM((NPAD,), _f32),
            pltpu.VMEM((NPAD,), _f32),
            pltpu.VMEM((32,), _f32),
            pltpu.VMEM((CN,), _f32),
            pltpu.VMEM((CN,), _f32),
            pltpu.VMEM_SHARED((NPAD,), _f32),
            pltpu.SemaphoreType.DMA,
        ],
    )
    def _round_kernel(src2, dst3, d_in, hprev, SAin, SBin, wb, SA, SB, hout,
                      src_v, dst2_v, msg_v, d_v, sa_v, sb_v, h_v, u_v,
                      wb_v, zero_v, out_v, agg_sp, sem):
        cid, sid, wid = _worker_ids()
        _stage(
            [
                (src2.at[wid], src_v),
                (dst3.at[wid], dst2_v),
                (d_in, d_v),
                (hprev, h_v),
                (SAin, sa_v),
                (SBin, sb_v),
                (wb, wb_v),
            ],
            sem,
        )
        _zero_and_barrier(zero_v, agg_sp, sid)
        wv = wb_v[pl.ds(0, 16)]
        bv = wb_v[pl.ds(16, 16)]

        @plsc.parallel_loop(0, NPAD // 16, unroll=8)
        def _(i):
            o = pl.ds(i * 16, 16)
            dd = d_v[o]
            hp = h_v[o]
            s = sa_v[o] + sb_v[o]
            z = (1.0 - ALPHA) * (dd * s + dd * dd * hp) + ALPHA * hp
            hr = z * wv + bv
            h_v[o] = hr
            u_v[o] = dd * hr

        chunk = pl.ds(sid * CN, CN)

        @pl.when(cid == 0)
        def _():
            pltpu.sync_copy(h_v.at[chunk], hout.at[chunk])

        _gather_msgs(src_v, u_v, msg_v)
        _scatter_chunks(msg_v, dst2_v, agg_sp, sem)
        _readback(agg_sp, out_v, SA, SB, cid, sid)

    return _round_kernel


_round_kernel = _make_round()


@functools.partial(
    pl.kernel,
    out_type=_sds((NPAD,), _f32),
    mesh=_mesh,
    compiler_params=_sc_params,
    scratch_types=[
        pltpu.VMEM((FW,), _f32),
        pltpu.VMEM((FW,), _f32),
        pltpu.VMEM((FW,), _f32),
        pltpu.VMEM((FW,), _f32),
        pltpu.VMEM((FW,), _f32),
    ],
)
def _final_kernel(SA, SB, d_in, h_in, y, sa_v, sb_v, d_v, h_v, y_v):
    cid, sid, wid = _worker_ids()
    chunk = pl.ds(wid * FW, FW)
    pltpu.sync_copy(
        (SA.at[chunk], SB.at[chunk], d_in.at[chunk], h_in.at[chunk]),
        (sa_v, sb_v, d_v, h_v),
    )

    @plsc.parallel_loop(0, FW // 16, unroll=4)
    def _(i):
        o = pl.ds(i * 16, 16)
        dd = d_v[o]
        hp = h_v[o]
        s = sa_v[o] + sb_v[o]
        z = (1.0 - ALPHA) * (dd * s + dd * dd * hp) + ALPHA * hp
        y_v[o] = jnp.maximum(z, 0.0) + 0.001

    pltpu.sync_copy(y_v, y.at[chunk])


ER = 2500
ERP = 2560
RF = ERP // NS

_mesh1 = plsc.VectorSubcoreMesh(
    core_axis_name="c", subcore_axis_name="s", num_cores=1
)


def _h0_body(x_ref, w_ref, b_ref, o_ref):
    o_ref[...] = jnp.sum(x_ref[...] * w_ref[...], axis=1) + b_ref[0]


def _h0_tc(x, W0r, b0):
    blk = NPAD // 10
    return pl.pallas_call(
        _h0_body,
        grid=(10,),
        in_specs=[
            pl.BlockSpec((blk, D), lambda i: (i, 0)),
            pl.BlockSpec((1, D), lambda i: (0, 0)),
            pl.BlockSpec(memory_space=pltpu.SMEM),
        ],
        out_specs=pl.BlockSpec((blk,), lambda i: (i,)),
        out_shape=_sds((NPAD,), _f32),
    )(x, W0r, b0)


def _scatter_rows(val2_v, dst2_v, agg_sp, sem, lo_row, n_rows, group):
    def grp(g, _):
        base = lo_row + g * group
        descs = [
            pltpu.async_copy(
                val2_v.at[base + j],
                agg_sp.at[dst2_v.at[base + j]],
                sem,
                add=True,
            )
            for j in range(group)
        ]
        for dsc in descs:
            dsc.wait()
        return 0

    lax.fori_loop(0, n_rows // group, grp, 0)


def _scatter_ones_rows(ones_v, dst2_v, agg_sp, sem, lo_row, n_rows, group):
    def grp(g, _):
        base = lo_row + g * group
        descs = [
            pltpu.async_copy(
                ones_v, agg_sp.at[dst2_v.at[base + j]], sem, add=True
            )
            for j in range(group)
        ]
        for dsc in descs:
            dsc.wait()
        return 0

    lax.fori_loop(0, n_rows // group, grp, 0)


def _gather_rows(src2_v, u_v, msg2_v, lo_row, n_rows, unroll):
    @plsc.parallel_loop(lo_row, lo_row + n_rows, unroll=unroll)
    def _(r):
        for c in range(8):
            o = pl.ds(c * 16, 16)
            msg2_v[r, o] = plsc.load_gather(u_v, [src2_v[r, o]])


@functools.partial(
    pl.kernel,
    out_type=(_sds((N,), _f32), _sds((NPAD,), _f32)),
    mesh=_mesh1,
    compiler_params=_sc_params,
    scratch_types=[
        pltpu.VMEM((RF, SCHUNK), jnp.int32),
        pltpu.VMEM((RF, SCHUNK), jnp.int32),
        pltpu.VMEM((RF, SCHUNK), _f32),
        pltpu.VMEM((NPAD,), _f32),
        pltpu.VMEM((SCHUNK,), _f32),
        pltpu.VMEM((CN,), _f32),
        pltpu.VMEM((CN,), _f32),
        pltpu.VMEM((CN,), _f32),
        pltpu.VMEM((CN,), _f32),
        pltpu.VMEM((64,), _f32),
        pltpu.VMEM_SHARED((NPAD,), _f32),
        pltpu.SemaphoreType.DMA,
    ],
)
def _mega2_kernel(ei3, h0p, wb, y, u_hbm,
                  src2_v, dst2_v, msg2_v, u_v, ones_v, d_c, h_c, s_c,
                  zero_v, wb_v, agg_sp, sem):
    sid = lax.axis_index("s")
    chunk = pl.ds(sid * CN, CN)
    rowbase = sid * RF

    _stage(
        [
            (ei3.at[0].at[pl.ds(rowbase, RF)], src2_v),
            (ei3.at[1].at[pl.ds(rowbase, RF)], dst2_v),
            (h0p.at[chunk], h_c),
            (wb, wb_v),
        ],
        sem,
    )
    _fill(ones_v, SCHUNK // 16, 1.0)
    _fill(zero_v, CN // 16, 0.0)
    pltpu.sync_copy(zero_v, agg_sp.at[chunk])
    plsc.subcore_barrier()

    _scatter_ones_rows(ones_v, dst2_v, agg_sp, sem, 0, RF, 10)
    plsc.subcore_barrier()
    pltpu.sync_copy(agg_sp.at[chunk], s_c)

    @plsc.parallel_loop(0, CN // 16, unroll=4)
    def _(i):
        o = pl.ds(i * 16, 16)
        dd = _rsqrt16(s_c[o] + 1.0)
        d_c[o] = dd
        s_c[o] = dd * h_c[o]

    pltpu.sync_copy(s_c, u_hbm.at[chunk])
    pltpu.sync_copy(zero_v, agg_sp.at[chunk])
    plsc.subcore_barrier()

    for r in (1, 2, 3):
        pltpu.sync_copy(u_hbm, u_v)
        _gather_rows(src2_v, u_v, msg2_v, 0, RF, 2)
        _scatter_rows(msg2_v, dst2_v, agg_sp, sem, 0, RF, 10)
        plsc.subcore_barrier()
        pltpu.sync_copy(agg_sp.at[chunk], s_c)

        if r < 3:
            wv = wb_v[pl.ds((r - 1) * 32, 16)]
            bv = wb_v[pl.ds((r - 1) * 32 + 16, 16)]

            @plsc.parallel_loop(0, CN // 16, unroll=4)
            def _(i):
                o = pl.ds(i * 16, 16)
                dd = d_c[o]
                hp = h_c[o]
                z = (1.0 - ALPHA) * (dd * s_c[o] + dd * dd * hp) + ALPHA * hp
                hr = z * wv + bv
                h_c[o] = hr
                s_c[o] = dd * hr

            pltpu.sync_copy(s_c, u_hbm.at[chunk])
            pltpu.sync_copy(zero_v, agg_sp.at[chunk])
            plsc.subcore_barrier()
        else:

            @plsc.parallel_loop(0, CN // 16, unroll=4)
            def _(i):
                o = pl.ds(i * 16, 16)
                dd = d_c[o]
                hp = h_c[o]
                z = (1.0 - ALPHA) * (dd * s_c[o] + dd * dd * hp) + ALPHA * hp
                s_c[o] = jnp.maximum(z, 0.0) + 0.001

            @pl.when(sid < 15)
            def _():
                pltpu.sync_copy(s_c, y.at[pl.ds(sid * CN, CN)])

            @pl.when(sid == 15)
            def _():
                pltpu.sync_copy(
                    s_c.at[pl.ds(0, N - 15 * CN)],
                    y.at[pl.ds(15 * CN, N - 15 * CN)],
                )


EW1 = EPAD // NS
NK1 = EW1 // SCHUNK

_mesh1 = plsc.VectorSubcoreMesh(
    core_axis_name="c", subcore_axis_name="s", num_cores=1
)


def _scatter_grouped(msg_v, dst2_v, agg_sp, sem, nk):
    G = 8

    def group(g, _):
        base = g * G
        descs = [
            pltpu.async_copy(
                msg_v.at[pl.ds((base + j) * SCHUNK, SCHUNK)],
                agg_sp.at[dst2_v.at[base + j]],
                sem,
                add=True,
            )
            for j in range(G)
        ]
        for dsc in descs:
            dsc.wait()
        return 0

    lax.fori_loop(0, nk // G, group, 0)


def _scatter_ones_grouped(ones_v, dst2_v, agg_sp, sem, nk):
    G = 8

    def group(g, _):
        base = g * G
        descs = [
            pltpu.async_copy(
                ones_v, agg_sp.at[dst2_v.at[base + j]], sem, add=True
            )
            for j in range(G)
        ]
        for dsc in descs:
            dsc.wait()
        return 0

    lax.fori_loop(0, nk // G, group, 0)


@functools.partial(
    pl.kernel,
    out_type=(_sds((NPAD,), _f32), _sds((NPAD,), _f32)),
    mesh=_mesh1,
    compiler_params=_sc_params,
    scratch_types=[
        pltpu.VMEM((EW1,), jnp.int32),
        pltpu.VMEM((NK1, SCHUNK), jnp.int32),
        pltpu.VMEM((EW1,), _f32),
        pltpu.VMEM((NPAD,), _f32),
        pltpu.VMEM((SCHUNK,), _f32),
        pltpu.VMEM((CN,), _f32),
        pltpu.VMEM((CN,), _f32),
        pltpu.VMEM((CN,), _f32),
        pltpu.VMEM((CN,), _f32),
        pltpu.VMEM((64,), _f32),
        pltpu.VMEM_SHARED((NPAD,), _f32),
        pltpu.SemaphoreType.DMA,
    ],
)
def _mega_kernel(src2, dst3, h0p, wb, y, u_hbm,
                 src_v, dst2_v, msg_v, u_v, ones_v, d_c, h_c, s_c,
                 zero_v, wb_v, agg_sp, sem):
    sid = lax.axis_index("s")
    chunk = pl.ds(sid * CN, CN)
    _stage(
        [
            (src2.at[sid], src_v),
            (dst3.at[sid], dst2_v),
            (h0p.at[chunk], h_c),
            (wb, wb_v),
        ],
        sem,
    )
    _fill(ones_v, SCHUNK // 16, 1.0)
    _fill(zero_v, CN // 16, 0.0)
    pltpu.sync_copy(zero_v, agg_sp.at[chunk])
    plsc.subcore_barrier()

    _scatter_ones_grouped(ones_v, dst2_v, agg_sp, sem, NK1)
    plsc.subcore_barrier()
    pltpu.sync_copy(agg_sp.at[chunk], s_c)

    @plsc.parallel_loop(0, CN // 16, unroll=4)
    def _(i):
        o = pl.ds(i * 16, 16)
        dd = _rsqrt16(s_c[o] + 1.0)
        d_c[o] = dd
        s_c[o] = dd * h_c[o]

    pltpu.sync_copy(s_c, u_hbm.at[chunk])
    pltpu.sync_copy(zero_v, agg_sp.at[chunk])
    plsc.subcore_barrier()

    for r in (1, 2, 3):
        pltpu.sync_copy(u_hbm, u_v)

        @plsc.parallel_loop(0, EW1 // 16, unroll=8)
        def _(i):
            o = pl.ds(i * 16, 16)
            msg_v[o] = plsc.load_gather(u_v, [src_v[o]])

        _scatter_grouped(msg_v, dst2_v, agg_sp, sem, NK1)
        plsc.subcore_barrier()
        pltpu.sync_copy(agg_sp.at[chunk], s_c)

        if r < 3:
            wv = wb_v[pl.ds((r - 1) * 32, 16)]
            bv = wb_v[pl.ds((r - 1) * 32 + 16, 16)]

            @plsc.parallel_loop(0, CN // 16, unroll=4)
            def _(i):
                o = pl.ds(i * 16, 16)
                dd = d_c[o]
                hp = h_c[o]
                z = (1.0 - ALPHA) * (dd * s_c[o] + dd * dd * hp) + ALPHA * hp
                hr = z * wv + bv
                h_c[o] = hr
                s_c[o] = dd * hr

            pltpu.sync_copy(s_c, u_hbm.at[chunk])
            pltpu.sync_copy(zero_v, agg_sp.at[chunk])
            plsc.subcore_barrier()
        else:

            @plsc.parallel_loop(0, CN // 16, unroll=4)
            def _(i):
                o = pl.ds(i * 16, 16)
                dd = d_c[o]
                hp = h_c[o]
                z = (1.0 - ALPHA) * (dd * s_c[o] + dd * dd * hp) + ALPHA * hp
                s_c[o] = jnp.maximum(z, 0.0) + 0.001

            pltpu.sync_copy(s_c, y.at[chunk])


def _wb_vec(W, b):
    return jnp.concatenate(
        [
            jnp.broadcast_to(W.reshape(-1)[:1], (16,)),
            jnp.broadcast_to(b.reshape(-1)[:1], (16,)),
        ]
    ).astype(_f32)


def kernel(x, edge_index, W0, b0, W1, b1, W2, b2):
    padc = (jnp.arange((ERP - ER) * SCHUNK, dtype=jnp.int32) % (NPAD - N) + N
            ).reshape(1, ERP - ER, SCHUNK)
    ei3 = jnp.concatenate(
        [
            edge_index.astype(jnp.int32).reshape(2, ER, SCHUNK),
            jnp.broadcast_to(padc, (2, ERP - ER, SCHUNK)),
        ],
        axis=1,
    )
    h0p = _h0_tc(x, W0.reshape(1, D), b0)
    wb = jnp.concatenate([_wb_vec(W1, b1), _wb_vec(W2, b2)])
    y, _unused = _mega2_kernel(ei3, h0p, wb)
    return y.reshape(N, 1)


# speedup vs baseline: 177.5319x; 1.1088x over previous
import functools

import jax
import jax.numpy as jnp
from jax import lax
from jax.experimental import pallas as pl
from jax.experimental.pallas import tpu as pltpu
from jax.experimental.pallas import tpu_sc as plsc

N = 10000
NPAD = 10240
E = 320000
D = 128
ALPHA = 0.2
NC = 2
NS = 16
NW = NC * NS
EPAD = NW * 10240
EW = EPAD // NW
SCHUNK = 128
NK = EW // SCHUNK
CN = NPAD // NS
FW = NPAD // NW

_mesh = plsc.VectorSubcoreMesh(core_axis_name="c", subcore_axis_name="s")
_sc_params = pltpu.CompilerParams(needs_layout_passes=False)

_f32 = jnp.float32
_sds = jax.ShapeDtypeStruct


def _worker_ids():
    cid = lax.axis_index("c")
    sid = lax.axis_index("s")
    return cid, sid, sid * NC + cid


def _fill(ref, n_vecs, value):
    v = jnp.full((16,), value, _f32)

    @plsc.parallel_loop(0, n_vecs, unroll=4)
    def _(i):
        ref[pl.ds(i * 16, 16)] = v


def _stage(pairs, sem):
    descs = [pltpu.async_copy(s, d, sem) for s, d in pairs]
    for dsc in descs:
        dsc.wait()


def _scatter_chunks(src_val_ref, dst2_v, agg_sp, sem):
    descs = []
    for k in range(NK):
        descs.append(
            pltpu.async_copy(
                src_val_ref.at[pl.ds(k * SCHUNK, SCHUNK)],
                agg_sp.at[dst2_v.at[k]],
                sem,
                add=True,
            )
        )
    for dsc in descs:
        dsc.wait()


def _zero_and_barrier(zero_v, agg_sp, sid):
    _fill(zero_v, CN // 16, 0.0)
    pltpu.sync_copy(zero_v, agg_sp.at[pl.ds(sid * CN, CN)])
    plsc.subcore_barrier()


def _readback(agg_sp, out_v, outA, outB, cid, sid):
    plsc.subcore_barrier()
    chunk = pl.ds(sid * CN, CN)
    pltpu.sync_copy(agg_sp.at[chunk], out_v)

    @pl.when(cid == 0)
    def _():
        pltpu.sync_copy(out_v, outA.at[chunk])

    @pl.when(cid == 1)
    def _():
        pltpu.sync_copy(out_v, outB.at[chunk])


def _rsqrt16(x):
    i = lax.bitcast_convert_type(x, jnp.int32)
    i = jnp.int32(0x5F3759DF) - (i >> 1)
    y = lax.bitcast_convert_type(i, _f32)
    y = y * (1.5 - 0.5 * x * y * y)
    y = y * (1.5 - 0.5 * x * y * y)
    y = y * (1.5 - 0.5 * x * y * y)
    return y


@functools.partial(
    pl.kernel,
    out_type=(_sds((NPAD,), _f32), _sds((NPAD,), _f32)),
    mesh=_mesh,
    compiler_params=_sc_params,
    scratch_types=[
        pltpu.VMEM((NK, SCHUNK), jnp.int32),
        pltpu.VMEM((SCHUNK,), _f32),
        pltpu.VMEM((CN,), _f32),
        pltpu.VMEM((CN,), _f32),
        pltpu.VMEM_SHARED((NPAD,), _f32),
        pltpu.SemaphoreType.DMA,
    ],
)
def _deg_kernel(dst3, degA, degB, dst2_v, ones_v, zero_v, out_v, agg_sp, sem):
    cid, sid, wid = _worker_ids()
    pltpu.sync_copy(dst3.at[wid], dst2_v)
    _fill(ones_v, SCHUNK // 16, 1.0)
    _zero_and_barrier(zero_v, agg_sp, sid)
    descs = [
        pltpu.async_copy(ones_v, agg_sp.at[dst2_v.at[k]], sem, add=True)
        for k in range(NK)
    ]
    for dsc in descs:
        dsc.wait()
    _readback(agg_sp, out_v, degA, degB, cid, sid)


def _h0_body(x_ref, w_ref, b_ref, o_ref):
    o_ref[...] = (
        jnp.dot(x_ref[...], w_ref[...], preferred_element_type=_f32) + b_ref[0]
    )


def _h0_tc(x, W0, b0):
    blk = 1000
    return pl.pallas_call(
        _h0_body,
        grid=(N // blk,),
        in_specs=[
            pl.BlockSpec((blk, D), lambda i: (i, 0)),
            pl.BlockSpec((D, 1), lambda i: (0, 0)),
            pl.BlockSpec(memory_space=pltpu.SMEM),
        ],
        out_specs=pl.BlockSpec((blk, 1), lambda i: (i, 0)),
        out_shape=_sds((N, 1), _f32),
    )(x, W0, b0)


def _gather_msgs(src_v, u_v, msg_v):
    @plsc.parallel_loop(0, EW // 16, unroll=8)
    def _(i):
        o = pl.ds(i * 16, 16)
        msg_v[o] = plsc.load_gather(u_v, [src_v[o]])


@functools.partial(
    pl.kernel,
    out_type=(_sds((NPAD,), _f32), _sds((NPAD,), _f32), _sds((NPAD,), _f32)),
    mesh=_mesh,
    compiler_params=_sc_params,
    scratch_types=[
        pltpu.VMEM((EW,), jnp.int32),
        pltpu.VMEM((NK, SCHUNK), jnp.int32),
        pltpu.VMEM((EW,), _f32),
        pltpu.VMEM((NPAD,), _f32),
        pltpu.VMEM((NPAD,), _f32),
        pltpu.VMEM((NPAD,), _f32),
        pltpu.VMEM((NPAD,), _f32),
        pltpu.VMEM((CN,), _f32),
        pltpu.VMEM((CN,), _f32),
        pltpu.VMEM_SHARED((NPAD,), _f32),
        pltpu.SemaphoreType.DMA,
    ],
)
def _round1_kernel(src2, dst3, degA, degB, h0p, SA, SB, dout,
                   src_v, dst2_v, msg_v, da_v, db_v, h_v, u_v,
                   zero_v, out_v, agg_sp, sem):
    cid, sid, wid = _worker_ids()
    _stage(
        [
            (src2.at[wid], src_v),
            (dst3.at[wid], dst2_v),
            (degA, da_v),
            (degB, db_v),
            (h0p, h_v),
        ],
        sem,
    )
    _zero_and_barrier(zero_v, agg_sp, sid)

    @plsc.parallel_loop(0, NPAD // 16, unroll=8)
    def _(i):
        o = pl.ds(i * 16, 16)
        deg = da_v[o] + db_v[o] + 1.0
        dd = _rsqrt16(deg)
        da_v[o] = dd
        u_v[o] = dd * h_v[o]

    chunk = pl.ds(sid * CN, CN)

    @pl.when(cid == 0)
    def _():
        pltpu.sync_copy(da_v.at[chunk], dout.at[chunk])

    _gather_msgs(src_v, u_v, msg_v)
    _scatter_chunks(msg_v, dst2_v, agg_sp, sem)
    _readback(agg_sp, out_v, SA, SB, cid, sid)


def _make_round():
    @functools.partial(
        pl.kernel,
        out_type=(_sds((NPAD,), _f32), _sds((NPAD,), _f32), _sds((NPAD,), _f32)),
        mesh=_mesh,
    compiler_params=_sc_params,
        scratch_types=[
            pltpu.VMEM((EW,), jnp.int32),
            pltpu.VMEM((NK, SCHUNK), jnp.int32),
            pltpu.VMEM((EW,), _f32),
            pltpu.VMEM((NPAD,), _f32),
            pltpu.VMEM((NPAD,), _f32),
            pltpu.VMEM((NPAD,), _f32),
            pltpu.VMEM((NPAD,), _f32),
            pltpu.VMEM((NPAD,), _f32),
            pltpu.VMEM((32,), _f32),
            pltpu.VMEM((CN,), _f32),
            pltpu.VMEM((CN,), _f32),
            pltpu.VMEM_SHARED((NPAD,), _f32),
            pltpu.SemaphoreType.DMA,
        ],
    )
    def _round_kernel(src2, dst3, d_in, hprev, SAin, SBin, wb, SA, SB, hout,
                      src_v, dst2_v, msg_v, d_v, sa_v, sb_v, h_v, u_v,
                      wb_v, zero_v, out_v, agg_sp, sem):
        cid, sid, wid = _worker_ids()
        _stage(
            [
                (src2.at[wid], src_v),
                (dst3.at[wid], dst2_v),
                (d_in, d_v),
                (hprev, h_v),
                (SAin, sa_v),
                (SBin, sb_v),
                (wb, wb_v),
            ],
            sem,
        )
        _zero_and_barrier(zero_v, agg_sp, sid)
        wv = wb_v[pl.ds(0, 16)]
        bv = wb_v[pl.ds(16, 16)]

        @plsc.parallel_loop(0, NPAD // 16, unroll=8)
        def _(i):
            o = pl.ds(i * 16, 16)
            dd = d_v[o]
            hp = h_v[o]
            s = sa_v[o] + sb_v[o]
            z = (1.0 - ALPHA) * (dd * s + dd * dd * hp) + ALPHA * hp
            hr = z * wv + bv
            h_v[o] = hr
            u_v[o] = dd * hr

        chunk = pl.ds(sid * CN, CN)

        @pl.when(cid == 0)
        def _():
            pltpu.sync_copy(h_v.at[chunk], hout.at[chunk])

        _gather_msgs(src_v, u_v, msg_v)
        _scatter_chunks(msg_v, dst2_v, agg_sp, sem)
        _readback(agg_sp, out_v, SA, SB, cid, sid)

    return _round_kernel


_round_kernel = _make_round()


@functools.partial(
    pl.kernel,
    out_type=_sds((NPAD,), _f32),
    mesh=_mesh,
    compiler_params=_sc_params,
    scratch_types=[
        pltpu.VMEM((FW,), _f32),
        pltpu.VMEM((FW,), _f32),
        pltpu.VMEM((FW,), _f32),
        pltpu.VMEM((FW,), _f32),
        pltpu.VMEM((FW,), _f32),
    ],
)
def _final_kernel(SA, SB, d_in, h_in, y, sa_v, sb_v, d_v, h_v, y_v):
    cid, sid, wid = _worker_ids()
    chunk = pl.ds(wid * FW, FW)
    pltpu.sync_copy(
        (SA.at[chunk], SB.at[chunk], d_in.at[chunk], h_in.at[chunk]),
        (sa_v, sb_v, d_v, h_v),
    )

    @plsc.parallel_loop(0, FW // 16, unroll=4)
    def _(i):
        o = pl.ds(i * 16, 16)
        dd = d_v[o]
        hp = h_v[o]
        s = sa_v[o] + sb_v[o]
        z = (1.0 - ALPHA) * (dd * s + dd * dd * hp) + ALPHA * hp
        y_v[o] = jnp.maximum(z, 0.0) + 0.001

    pltpu.sync_copy(y_v, y.at[chunk])


ER = 2500
ERP = 2560
RF = ERP // NS

_mesh1 = plsc.VectorSubcoreMesh(
    core_axis_name="c", subcore_axis_name="s", num_cores=1
)


def _h0_body(x_ref, w_ref, b_ref, o_ref):
    o_ref[...] = jnp.sum(x_ref[...] * w_ref[...], axis=1) + b_ref[0]


def _h0_tc(x, W0r, b0):
    blk = NPAD // 10
    return pl.pallas_call(
        _h0_body,
        grid=(10,),
        in_specs=[
            pl.BlockSpec((blk, D), lambda i: (i, 0)),
            pl.BlockSpec((1, D), lambda i: (0, 0)),
            pl.BlockSpec(memory_space=pltpu.SMEM),
        ],
        out_specs=pl.BlockSpec((blk,), lambda i: (i,)),
        out_shape=_sds((NPAD,), _f32),
    )(x, W0r, b0)


def _scatter_rows(val2_v, dst2_v, agg_sp, sem, lo_row, n_rows, group):
    def grp(g, _):
        base = lo_row + g * group
        descs = [
            pltpu.async_copy(
                val2_v.at[base + j],
                agg_sp.at[dst2_v.at[base + j]],
                sem,
                add=True,
            )
            for j in range(group)
        ]
        for dsc in descs:
            dsc.wait()
        return 0

    lax.fori_loop(0, n_rows // group, grp, 0)


def _scatter_ones_rows(ones_v, dst2_v, agg_sp, sem, lo_row, n_rows, group):
    def grp(g, _):
        base = lo_row + g * group
        descs = [
            pltpu.async_copy(
                ones_v, agg_sp.at[dst2_v.at[base + j]], sem, add=True
            )
            for j in range(group)
        ]
        for dsc in descs:
            dsc.wait()
        return 0

    lax.fori_loop(0, n_rows // group, grp, 0)


def _gather_rows(src2_v, u_v, msg2_v, lo_row, n_rows, unroll):
    @plsc.parallel_loop(lo_row, lo_row + n_rows, unroll=unroll)
    def _(r):
        for c in range(8):
            o = pl.ds(c * 16, 16)
            msg2_v[r, o] = plsc.load_gather(u_v, [src2_v[r, o]])


@functools.partial(
    pl.kernel,
    out_type=(_sds((N,), _f32), _sds((NPAD,), _f32)),
    mesh=_mesh1,
    compiler_params=_sc_params,
    scratch_types=[
        pltpu.VMEM((RF, SCHUNK), jnp.int32),
        pltpu.VMEM((RF, SCHUNK), jnp.int32),
        pltpu.VMEM((RF, SCHUNK), _f32),
        pltpu.VMEM((NPAD,), _f32),
        pltpu.VMEM((SCHUNK,), _f32),
        pltpu.VMEM((CN,), _f32),
        pltpu.VMEM((CN,), _f32),
        pltpu.VMEM((CN,), _f32),
        pltpu.VMEM((CN,), _f32),
        pltpu.VMEM((64,), _f32),
        pltpu.VMEM_SHARED((NPAD,), _f32),
        pltpu.SemaphoreType.DMA,
    ],
)
def _mega2_kernel(ei3, h0p, wb, y, u_hbm,
                  src2_v, dst2_v, msg2_v, u_v, ones_v, d_c, h_c, s_c,
                  zero_v, wb_v, agg_sp, sem):
    sid = lax.axis_index("s")
    chunk = pl.ds(sid * CN, CN)
    rowbase = sid * RF

    _stage(
        [
            (ei3.at[0].at[pl.ds(rowbase, RF)], src2_v),
            (ei3.at[1].at[pl.ds(rowbase, RF)], dst2_v),
            (h0p.at[chunk], h_c),
            (wb, wb_v),
        ],
        sem,
    )
    _fill(ones_v, SCHUNK // 16, 1.0)
    _fill(zero_v, CN // 16, 0.0)
    pltpu.sync_copy(zero_v, agg_sp.at[chunk])
    plsc.subcore_barrier()

    _scatter_ones_rows(ones_v, dst2_v, agg_sp, sem, 0, RF, 10)
    plsc.subcore_barrier()
    pltpu.sync_copy(agg_sp.at[chunk], s_c)

    @plsc.parallel_loop(0, CN // 16, unroll=4)
    def _(i):
        o = pl.ds(i * 16, 16)
        dd = _rsqrt16(s_c[o] + 1.0)
        d_c[o] = dd
        s_c[o] = dd * h_c[o]

    pltpu.sync_copy(s_c, u_hbm.at[chunk])
    pltpu.sync_copy(zero_v, agg_sp.at[chunk])
    plsc.subcore_barrier()

    for r in (1, 2, 3):
        pltpu.sync_copy(u_hbm, u_v)
        G2 = 8

        def gs(g, _):
            base = g * G2

            @plsc.parallel_loop(base, base + G2, unroll=2)
            def _(r):
                for c in range(8):
                    o = pl.ds(c * 16, 16)
                    msg2_v[r, o] = plsc.load_gather(u_v, [src2_v[r, o]])

            for j in range(G2):
                pltpu.async_copy(
                    msg2_v.at[base + j],
                    agg_sp.at[dst2_v.at[base + j]],
                    sem,
                    add=True,
                )
            return 0

        lax.fori_loop(0, RF // G2, gs, 0)

        def drain(g, _):
            base = g * G2
            for j in range(G2):
                pltpu.make_async_copy(
                    msg2_v.at[base + j],
                    agg_sp.at[dst2_v.at[base + j]],
                    sem,
                ).wait()
            return 0

        lax.fori_loop(0, RF // G2, drain, 0)
        plsc.subcore_barrier()
        pltpu.sync_copy(agg_sp.at[chunk], s_c)

        if r < 3:
            wv = wb_v[pl.ds((r - 1) * 32, 16)]
            bv = wb_v[pl.ds((r - 1) * 32 + 16, 16)]

            @plsc.parallel_loop(0, CN // 16, unroll=4)
            def _(i):
                o = pl.ds(i * 16, 16)
                dd = d_c[o]
                hp = h_c[o]
                z = (1.0 - ALPHA) * (dd * s_c[o] + dd * dd * hp) + ALPHA * hp
                hr = z * wv + bv
                h_c[o] = hr
                s_c[o] = dd * hr

            pltpu.sync_copy(s_c, u_hbm.at[chunk])
            pltpu.sync_copy(zero_v, agg_sp.at[chunk])
            plsc.subcore_barrier()
        else:

            @plsc.parallel_loop(0, CN // 16, unroll=4)
            def _(i):
                o = pl.ds(i * 16, 16)
                dd = d_c[o]
                hp = h_c[o]
                z = (1.0 - ALPHA) * (dd * s_c[o] + dd * dd * hp) + ALPHA * hp
                s_c[o] = jnp.maximum(z, 0.0) + 0.001

            @pl.when(sid < 15)
            def _():
                pltpu.sync_copy(s_c, y.at[pl.ds(sid * CN, CN)])

            @pl.when(sid == 15)
            def _():
                pltpu.sync_copy(
                    s_c.at[pl.ds(0, N - 15 * CN)],
                    y.at[pl.ds(15 * CN, N - 15 * CN)],
                )


EW1 = EPAD // NS
NK1 = EW1 // SCHUNK

_mesh1 = plsc.VectorSubcoreMesh(
    core_axis_name="c", subcore_axis_name="s", num_cores=1
)


def _scatter_grouped(msg_v, dst2_v, agg_sp, sem, nk):
    G = 8

    def group(g, _):
        base = g * G
        descs = [
            pltpu.async_copy(
                msg_v.at[pl.ds((base + j) * SCHUNK, SCHUNK)],
                agg_sp.at[dst2_v.at[base + j]],
                sem,
                add=True,
            )
            for j in range(G)
        ]
        for dsc in descs:
            dsc.wait()
        return 0

    lax.fori_loop(0, nk // G, group, 0)


def _scatter_ones_grouped(ones_v, dst2_v, agg_sp, sem, nk):
    G = 8

    def group(g, _):
        base = g * G
        descs = [
            pltpu.async_copy(
                ones_v, agg_sp.at[dst2_v.at[base + j]], sem, add=True
            )
            for j in range(G)
        ]
        for dsc in descs:
            dsc.wait()
        return 0

    lax.fori_loop(0, nk // G, group, 0)


@functools.partial(
    pl.kernel,
    out_type=(_sds((NPAD,), _f32), _sds((NPAD,), _f32)),
    mesh=_mesh1,
    compiler_params=_sc_params,
    scratch_types=[
        pltpu.VMEM((EW1,), jnp.int32),
        pltpu.VMEM((NK1, SCHUNK), jnp.int32),
        pltpu.VMEM((EW1,), _f32),
        pltpu.VMEM((NPAD,), _f32),
        pltpu.VMEM((SCHUNK,), _f32),
        pltpu.VMEM((CN,), _f32),
        pltpu.VMEM((CN,), _f32),
        pltpu.VMEM((CN,), _f32),
        pltpu.VMEM((CN,), _f32),
        pltpu.VMEM((64,), _f32),
        pltpu.VMEM_SHARED((NPAD,), _f32),
        pltpu.SemaphoreType.DMA,
    ],
)
def _mega_kernel(src2, dst3, h0p, wb, y, u_hbm,
                 src_v, dst2_v, msg_v, u_v, ones_v, d_c, h_c, s_c,
                 zero_v, wb_v, agg_sp, sem):
    sid = lax.axis_index("s")
    chunk = pl.ds(sid * CN, CN)
    _stage(
        [
            (src2.at[sid], src_v),
            (dst3.at[sid], dst2_v),
            (h0p.at[chunk], h_c),
            (wb, wb_v),
        ],
        sem,
    )
    _fill(ones_v, SCHUNK // 16, 1.0)
    _fill(zero_v, CN // 16, 0.0)
    pltpu.sync_copy(zero_v, agg_sp.at[chunk])
    plsc.subcore_barrier()

    _scatter_ones_grouped(ones_v, dst2_v, agg_sp, sem, NK1)
    plsc.subcore_barrier()
    pltpu.sync_copy(agg_sp.at[chunk], s_c)

    @plsc.parallel_loop(0, CN // 16, unroll=4)
    def _(i):
        o = pl.ds(i * 16, 16)
        dd = _rsqrt16(s_c[o] + 1.0)
        d_c[o] = dd
        s_c[o] = dd * h_c[o]

    pltpu.sync_copy(s_c, u_hbm.at[chunk])
    pltpu.sync_copy(zero_v, agg_sp.at[chunk])
    plsc.subcore_barrier()

    for r in (1, 2, 3):
        pltpu.sync_copy(u_hbm, u_v)

        @plsc.parallel_loop(0, EW1 // 16, unroll=8)
        def _(i):
            o = pl.ds(i * 16, 16)
            msg_v[o] = plsc.load_gather(u_v, [src_v[o]])

        _scatter_grouped(msg_v, dst2_v, agg_sp, sem, NK1)
        plsc.subcore_barrier()
        pltpu.sync_copy(agg_sp.at[chunk], s_c)

        if r < 3:
            wv = wb_v[pl.ds((r - 1) * 32, 16)]
            bv = wb_v[pl.ds((r - 1) * 32 + 16, 16)]

            @plsc.parallel_loop(0, CN // 16, unroll=4)
            def _(i):
                o = pl.ds(i * 16, 16)
                dd = d_c[o]
                hp = h_c[o]
                z = (1.0 - ALPHA) * (dd * s_c[o] + dd * dd * hp) + ALPHA * hp
                hr = z * wv + bv
                h_c[o] = hr
                s_c[o] = dd * hr

            pltpu.sync_copy(s_c, u_hbm.at[chunk])
            pltpu.sync_copy(zero_v, agg_sp.at[chunk])
            plsc.subcore_barrier()
        else:

            @plsc.parallel_loop(0, CN // 16, unroll=4)
            def _(i):
                o = pl.ds(i * 16, 16)
                dd = d_c[o]
                hp = h_c[o]
                z = (1.0 - ALPHA) * (dd * s_c[o] + dd * dd * hp) + ALPHA * hp
                s_c[o] = jnp.maximum(z, 0.0) + 0.001

            pltpu.sync_copy(s_c, y.at[chunk])


def _wb_vec(W, b):
    return jnp.concatenate(
        [
            jnp.broadcast_to(W.reshape(-1)[:1], (16,)),
            jnp.broadcast_to(b.reshape(-1)[:1], (16,)),
        ]
    ).astype(_f32)


def kernel(x, edge_index, W0, b0, W1, b1, W2, b2):
    padc = (jnp.arange((ERP - ER) * SCHUNK, dtype=jnp.int32) % (NPAD - N) + N
            ).reshape(1, ERP - ER, SCHUNK)
    ei3 = jnp.concatenate(
        [
            edge_index.astype(jnp.int32).reshape(2, ER, SCHUNK),
            jnp.broadcast_to(padc, (2, ERP - ER, SCHUNK)),
        ],
        axis=1,
    )
    h0p = _h0_tc(x, W0.reshape(1, D), b0)
    wb = jnp.concatenate([_wb_vec(W1, b1), _wb_vec(W2, b2)])
    y, _unused = _mega2_kernel(ei3, h0p, wb)
    return y.reshape(N, 1)


# speedup vs baseline: 179.6331x; 1.0118x over previous
import functools

import jax
import jax.numpy as jnp
from jax import lax
from jax.experimental import pallas as pl
from jax.experimental.pallas import tpu as pltpu
from jax.experimental.pallas import tpu_sc as plsc

N = 10000
NPAD = 10240
E = 320000
D = 128
ALPHA = 0.2
NC = 2
NS = 16
NW = NC * NS
EPAD = NW * 10240
EW = EPAD // NW
SCHUNK = 128
NK = EW // SCHUNK
CN = NPAD // NS
FW = NPAD // NW

_mesh = plsc.VectorSubcoreMesh(core_axis_name="c", subcore_axis_name="s")
_sc_params = pltpu.CompilerParams(needs_layout_passes=False)

_f32 = jnp.float32
_sds = jax.ShapeDtypeStruct


def _worker_ids():
    cid = lax.axis_index("c")
    sid = lax.axis_index("s")
    return cid, sid, sid * NC + cid


def _fill(ref, n_vecs, value):
    v = jnp.full((16,), value, _f32)

    @plsc.parallel_loop(0, n_vecs, unroll=4)
    def _(i):
        ref[pl.ds(i * 16, 16)] = v


def _stage(pairs, sem):
    descs = [pltpu.async_copy(s, d, sem) for s, d in pairs]
    for dsc in descs:
        dsc.wait()


def _scatter_chunks(src_val_ref, dst2_v, agg_sp, sem):
    descs = []
    for k in range(NK):
        descs.append(
            pltpu.async_copy(
                src_val_ref.at[pl.ds(k * SCHUNK, SCHUNK)],
                agg_sp.at[dst2_v.at[k]],
                sem,
                add=True,
            )
        )
    for dsc in descs:
        dsc.wait()


def _zero_and_barrier(zero_v, agg_sp, sid):
    _fill(zero_v, CN // 16, 0.0)
    pltpu.sync_copy(zero_v, agg_sp.at[pl.ds(sid * CN, CN)])
    plsc.subcore_barrier()


def _readback(agg_sp, out_v, outA, outB, cid, sid):
    plsc.subcore_barrier()
    chunk = pl.ds(sid * CN, CN)
    pltpu.sync_copy(agg_sp.at[chunk], out_v)

    @pl.when(cid == 0)
    def _():
        pltpu.sync_copy(out_v, outA.at[chunk])

    @pl.when(cid == 1)
    def _():
        pltpu.sync_copy(out_v, outB.at[chunk])


def _rsqrt16(x):
    i = lax.bitcast_convert_type(x, jnp.int32)
    i = jnp.int32(0x5F3759DF) - (i >> 1)
    y = lax.bitcast_convert_type(i, _f32)
    y = y * (1.5 - 0.5 * x * y * y)
    y = y * (1.5 - 0.5 * x * y * y)
    y = y * (1.5 - 0.5 * x * y * y)
    return y


@functools.partial(
    pl.kernel,
    out_type=(_sds((NPAD,), _f32), _sds((NPAD,), _f32)),
    mesh=_mesh,
    compiler_params=_sc_params,
    scratch_types=[
        pltpu.VMEM((NK, SCHUNK), jnp.int32),
        pltpu.VMEM((SCHUNK,), _f32),
        pltpu.VMEM((CN,), _f32),
        pltpu.VMEM((CN,), _f32),
        pltpu.VMEM_SHARED((NPAD,), _f32),
        pltpu.SemaphoreType.DMA,
    ],
)
def _deg_kernel(dst3, degA, degB, dst2_v, ones_v, zero_v, out_v, agg_sp, sem):
    cid, sid, wid = _worker_ids()
    pltpu.sync_copy(dst3.at[wid], dst2_v)
    _fill(ones_v, SCHUNK // 16, 1.0)
    _zero_and_barrier(zero_v, agg_sp, sid)
    descs = [
        pltpu.async_copy(ones_v, agg_sp.at[dst2_v.at[k]], sem, add=True)
        for k in range(NK)
    ]
    for dsc in descs:
        dsc.wait()
    _readback(agg_sp, out_v, degA, degB, cid, sid)


def _h0_body(x_ref, w_ref, b_ref, o_ref):
    o_ref[...] = (
        jnp.dot(x_ref[...], w_ref[...], preferred_element_type=_f32) + b_ref[0]
    )


def _h0_tc(x, W0, b0):
    blk = 1000
    return pl.pallas_call(
        _h0_body,
        grid=(N // blk,),
        in_specs=[
            pl.BlockSpec((blk, D), lambda i: (i, 0)),
            pl.BlockSpec((D, 1), lambda i: (0, 0)),
            pl.BlockSpec(memory_space=pltpu.SMEM),
        ],
        out_specs=pl.BlockSpec((blk, 1), lambda i: (i, 0)),
        out_shape=_sds((N, 1), _f32),
    )(x, W0, b0)


def _gather_msgs(src_v, u_v, msg_v):
    @plsc.parallel_loop(0, EW // 16, unroll=8)
    def _(i):
        o = pl.ds(i * 16, 16)
        msg_v[o] = plsc.load_gather(u_v, [src_v[o]])


@functools.partial(
    pl.kernel,
    out_type=(_sds((NPAD,), _f32), _sds((NPAD,), _f32), _sds((NPAD,), _f32)),
    mesh=_mesh,
    compiler_params=_sc_params,
    scratch_types=[
        pltpu.VMEM((EW,), jnp.int32),
        pltpu.VMEM((NK, SCHUNK), jnp.int32),
        pltpu.VMEM((EW,), _f32),
        pltpu.VMEM((NPAD,), _f32),
        pltpu.VMEM((NPAD,), _f32),
        pltpu.VMEM((NPAD,), _f32),
        pltpu.VMEM((NPAD,), _f32),
        pltpu.VMEM((CN,), _f32),
        pltpu.VMEM((CN,), _f32),
        pltpu.VMEM_SHARED((NPAD,), _f32),
        pltpu.SemaphoreType.DMA,
    ],
)
def _round1_kernel(src2, dst3, degA, degB, h0p, SA, SB, dout,
                   src_v, dst2_v, msg_v, da_v, db_v, h_v, u_v,
                   zero_v, out_v, agg_sp, sem):
    cid, sid, wid = _worker_ids()
    _stage(
        [
            (src2.at[wid], src_v),
            (dst3.at[wid], dst2_v),
            (degA, da_v),
            (degB, db_v),
            (h0p, h_v),
        ],
        sem,
    )
    _zero_and_barrier(zero_v, agg_sp, sid)

    @plsc.parallel_loop(0, NPAD // 16, unroll=8)
    def _(i):
        o = pl.ds(i * 16, 16)
        deg = da_v[o] + db_v[o] + 1.0
        dd = _rsqrt16(deg)
        da_v[o] = dd
        u_v[o] = dd * h_v[o]

    chunk = pl.ds(sid * CN, CN)

    @pl.when(cid == 0)
    def _():
        pltpu.sync_copy(da_v.at[chunk], dout.at[chunk])

    _gather_msgs(src_v, u_v, msg_v)
    _scatter_chunks(msg_v, dst2_v, agg_sp, sem)
    _readback(agg_sp, out_v, SA, SB, cid, sid)


def _make_round():
    @functools.partial(
        pl.kernel,
        out_type=(_sds((NPAD,), _f32), _sds((NPAD,), _f32), _sds((NPAD,), _f32)),
        mesh=_mesh,
    compiler_params=_sc_params,
        scratch_types=[
            pltpu.VMEM((EW,), jnp.int32),
            pltpu.VMEM((NK, SCHUNK), jnp.int32),
            pltpu.VMEM((EW,), _f32),
            pltpu.VMEM((NPAD,), _f32),
            pltpu.VMEM((NPAD,), _f32),
            pltpu.VMEM((NPAD,), _f32),
            pltpu.VMEM((NPAD,), _f32),
            pltpu.VMEM((NPAD,), _f32),
            pltpu.VMEM((32,), _f32),
            pltpu.VMEM((CN,), _f32),
            pltpu.VMEM((CN,), _f32),
            pltpu.VMEM_SHARED((NPAD,), _f32),
            pltpu.SemaphoreType.DMA,
        ],
    )
    def _round_kernel(src2, dst3, d_in, hprev, SAin, SBin, wb, SA, SB, hout,
                      src_v, dst2_v, msg_v, d_v, sa_v, sb_v, h_v, u_v,
                      wb_v, zero_v, out_v, agg_sp, sem):
        cid, sid, wid = _worker_ids()
        _stage(
            [
                (src2.at[wid], src_v),
                (dst3.at[wid], dst2_v),
                (d_in, d_v),
                (hprev, h_v),
                (SAin, sa_v),
                (SBin, sb_v),
                (wb, wb_v),
            ],
            sem,
        )
        _zero_and_barrier(zero_v, agg_sp, sid)
        wv = wb_v[pl.ds(0, 16)]
        bv = wb_v[pl.ds(16, 16)]

        @plsc.parallel_loop(0, NPAD // 16, unroll=8)
        def _(i):
            o = pl.ds(i * 16, 16)
            dd = d_v[o]
            hp = h_v[o]
            s = sa_v[o] + sb_v[o]
            z = (1.0 - ALPHA) * (dd * s + dd * dd * hp) + ALPHA * hp
            hr = z * wv + bv
            h_v[o] = hr
            u_v[o] = dd * hr

        chunk = pl.ds(sid * CN, CN)

        @pl.when(cid == 0)
        def _():
            pltpu.sync_copy(h_v.at[chunk], hout.at[chunk])

        _gather_msgs(src_v, u_v, msg_v)
        _scatter_chunks(msg_v, dst2_v, agg_sp, sem)
        _readback(agg_sp, out_v, SA, SB, cid, sid)

    return _round_kernel


_round_kernel = _make_round()


@functools.partial(
    pl.kernel,
    out_type=_sds((NPAD,), _f32),
    mesh=_mesh,
    compiler_params=_sc_params,
    scratch_types=[
        pltpu.VMEM((FW,), _f32),
        pltpu.VMEM((FW,), _f32),
        pltpu.VMEM((FW,), _f32),
        pltpu.VMEM((FW,), _f32),
        pltpu.VMEM((FW,), _f32),
    ],
)
def _final_kernel(SA, SB, d_in, h_in, y, sa_v, sb_v, d_v, h_v, y_v):
    cid, sid, wid = _worker_ids()
    chunk = pl.ds(wid * FW, FW)
    pltpu.sync_copy(
        (SA.at[chunk], SB.at[chunk], d_in.at[chunk], h_in.at[chunk]),
        (sa_v, sb_v, d_v, h_v),
    )

    @plsc.parallel_loop(0, FW // 16, unroll=4)
    def _(i):
        o = pl.ds(i * 16, 16)
        dd = d_v[o]
        hp = h_v[o]
        s = sa_v[o] + sb_v[o]
        z = (1.0 - ALPHA) * (dd * s + dd * dd * hp) + ALPHA * hp
        y_v[o] = jnp.maximum(z, 0.0) + 0.001

    pltpu.sync_copy(y_v, y.at[chunk])


ER = 2500
ERP = 2560
RF = ERP // NS

_mesh1 = plsc.VectorSubcoreMesh(
    core_axis_name="c", subcore_axis_name="s", num_cores=1
)


def _h0_body(x_ref, w_ref, b_ref, o_ref):
    o_ref[...] = jnp.sum(x_ref[...] * w_ref[...], axis=1) + b_ref[0]


def _h0_tc(x, W0r, b0):
    blk = NPAD // 10
    return pl.pallas_call(
        _h0_body,
        grid=(10,),
        in_specs=[
            pl.BlockSpec((blk, D), lambda i: (i, 0)),
            pl.BlockSpec((1, D), lambda i: (0, 0)),
            pl.BlockSpec(memory_space=pltpu.SMEM),
        ],
        out_specs=pl.BlockSpec((blk,), lambda i: (i,)),
        out_shape=_sds((NPAD,), _f32),
    )(x, W0r, b0)


def _scatter_rows(val2_v, dst2_v, agg_sp, sem, lo_row, n_rows, group):
    def grp(g, _):
        base = lo_row + g * group
        descs = [
            pltpu.async_copy(
                val2_v.at[base + j],
                agg_sp.at[dst2_v.at[base + j]],
                sem,
                add=True,
            )
            for j in range(group)
        ]
        for dsc in descs:
            dsc.wait()
        return 0

    lax.fori_loop(0, n_rows // group, grp, 0)


def _scatter_ones_rows(ones_v, dst2_v, agg_sp, sem, lo_row, n_rows, group):
    def grp(g, _):
        base = lo_row + g * group
        descs = [
            pltpu.async_copy(
                ones_v, agg_sp.at[dst2_v.at[base + j]], sem, add=True
            )
            for j in range(group)
        ]
        for dsc in descs:
            dsc.wait()
        return 0

    lax.fori_loop(0, n_rows // group, grp, 0)


def _gather_rows(src2_v, u_v, msg2_v, lo_row, n_rows, unroll):
    @plsc.parallel_loop(lo_row, lo_row + n_rows, unroll=unroll)
    def _(r):
        for c in range(8):
            o = pl.ds(c * 16, 16)
            msg2_v[r, o] = plsc.load_gather(u_v, [src2_v[r, o]])


@functools.partial(
    pl.kernel,
    out_type=(_sds((N,), _f32), _sds((NPAD,), _f32)),
    mesh=_mesh1,
    compiler_params=_sc_params,
    scratch_types=[
        pltpu.VMEM((RF, SCHUNK), jnp.int32),
        pltpu.VMEM((RF, SCHUNK), jnp.int32),
        pltpu.VMEM((RF, SCHUNK), _f32),
        pltpu.VMEM((NPAD,), _f32),
        pltpu.VMEM((SCHUNK,), _f32),
        pltpu.VMEM((CN,), _f32),
        pltpu.VMEM((CN,), _f32),
        pltpu.VMEM((CN,), _f32),
        pltpu.VMEM((CN,), _f32),
        pltpu.VMEM((64,), _f32),
        pltpu.VMEM_SHARED((NPAD,), _f32),
        pltpu.SemaphoreType.DMA,
    ],
)
def _mega2_kernel(ei3, h0p, wb, y, u_hbm,
                  src2_v, dst2_v, msg2_v, u_v, ones_v, d_c, h_c, s_c,
                  zero_v, wb_v, agg_sp, sem):
    sid = lax.axis_index("s")
    chunk = pl.ds(sid * CN, CN)
    rowbase = sid * RF

    _stage(
        [
            (ei3.at[0].at[pl.ds(rowbase, RF)], src2_v),
            (ei3.at[1].at[pl.ds(rowbase, RF)], dst2_v),
            (h0p.at[chunk], h_c),
            (wb, wb_v),
        ],
        sem,
    )
    _fill(ones_v, SCHUNK // 16, 1.0)
    _fill(zero_v, CN // 16, 0.0)
    pltpu.sync_copy(zero_v, agg_sp.at[chunk])
    plsc.subcore_barrier()

    _scatter_ones_rows(ones_v, dst2_v, agg_sp, sem, 0, RF, 10)
    plsc.subcore_barrier()
    pltpu.sync_copy(agg_sp.at[chunk], s_c)

    @plsc.parallel_loop(0, CN // 16, unroll=4)
    def _(i):
        o = pl.ds(i * 16, 16)
        dd = _rsqrt16(s_c[o] + 1.0)
        d_c[o] = dd
        s_c[o] = dd * h_c[o]

    pltpu.sync_copy(s_c, u_hbm.at[chunk])
    pltpu.sync_copy(zero_v, agg_sp.at[chunk])
    plsc.subcore_barrier()

    for r in (1, 2, 3):
        pltpu.sync_copy(u_hbm, u_v)
        G2 = 8

        def gs(g, _):
            base = g * G2

            @plsc.parallel_loop(base, base + G2, unroll=2)
            def _(r):
                for c in range(8):
                    o = pl.ds(c * 16, 16)
                    msg2_v[r, o] = plsc.load_gather(u_v, [src2_v[r, o]])

            for j in range(G2):
                pltpu.async_copy(
                    msg2_v.at[base + j],
                    agg_sp.at[dst2_v.at[base + j]],
                    sem,
                    add=True,
                )
            return 0

        lax.fori_loop(0, RF // G2, gs, 0)

        def drain(g, _):
            base = g * G2
            for j in range(G2):
                pltpu.make_async_copy(
                    msg2_v.at[base + j],
                    agg_sp.at[dst2_v.at[base + j]],
                    sem,
                ).wait()
            return 0

        lax.fori_loop(0, RF // G2, drain, 0)
        plsc.subcore_barrier()
        pltpu.sync_copy(agg_sp.at[chunk], s_c)

        if r < 3:
            wv = wb_v[pl.ds((r - 1) * 32, 16)]
            bv = wb_v[pl.ds((r - 1) * 32 + 16, 16)]

            @plsc.parallel_loop(0, CN // 16, unroll=4)
            def _(i):
                o = pl.ds(i * 16, 16)
                dd = d_c[o]
                hp = h_c[o]
                z = (1.0 - ALPHA) * (dd * s_c[o] + dd * dd * hp) + ALPHA * hp
                hr = z * wv + bv
                h_c[o] = hr
                s_c[o] = dd * hr

            pltpu.sync_copy(s_c, u_hbm.at[chunk])
            pltpu.sync_copy(zero_v, agg_sp.at[chunk])
            plsc.subcore_barrier()
        else:

            @plsc.parallel_loop(0, CN // 16, unroll=4)
            def _(i):
                o = pl.ds(i * 16, 16)
                dd = d_c[o]
                hp = h_c[o]
                z = (1.0 - ALPHA) * (dd * s_c[o] + dd * dd * hp) + ALPHA * hp
                s_c[o] = jnp.maximum(z, 0.0) + 0.001

            @pl.when(sid < 15)
            def _():
                pltpu.sync_copy(s_c, y.at[pl.ds(sid * CN, CN)])

            @pl.when(sid == 15)
            def _():
                pltpu.sync_copy(
                    s_c.at[pl.ds(0, N - 15 * CN)],
                    y.at[pl.ds(15 * CN, N - 15 * CN)],
                )


RF2 = ERP // NW

_mesh2 = plsc.VectorSubcoreMesh(core_axis_name="c", subcore_axis_name="s")


@functools.partial(
    pl.kernel,
    out_type=(
        [_sds((N,), _f32)]
        + [_sds((NPAD,), _f32)] * 3
        + [_sds((NPAD,), _f32)] * 8
        + [_sds((16,), _f32)] * 2
    ),
    mesh=_mesh2,
    compiler_params=_sc_params,
    scratch_types=[
        pltpu.VMEM((RF2, SCHUNK), jnp.int32),
        pltpu.VMEM((RF2, SCHUNK), jnp.int32),
        pltpu.VMEM((RF2, SCHUNK), _f32),
        pltpu.VMEM((NPAD,), _f32),
        pltpu.VMEM((SCHUNK,), _f32),
        pltpu.VMEM((CN,), _f32),
        pltpu.VMEM((CN,), _f32),
        pltpu.VMEM((CN,), _f32),
        pltpu.VMEM((CN,), _f32),
        pltpu.VMEM((CN,), _f32),
        pltpu.VMEM((64,), _f32),
        pltpu.VMEM((16,), _f32),
        pltpu.VMEM_SHARED((NPAD,), _f32),
        pltpu.SemaphoreType.DMA,
    ],
)
def _mega3_kernel(ei3, h0p, wb,
                  y, u0, u1, u2, sa0, sa1, sa2, sa3, sb0, sb1, sb2, sb3,
                  fA, fB,
                  src2_v, dst2_v, msg2_v, u_v, ones_v, d_c, h_c, s_c, o_c,
                  zero_v, wb_v, flag_v, agg_sp, sem):
    cid = lax.axis_index("c")
    sid = lax.axis_index("s")
    wid = cid * NS + sid
    chunk = pl.ds(sid * CN, CN)
    rowbase = wid * RF2
    u_bufs = (u0, u1, u2)
    exp_a = (sa0, sa1, sa2, sa3)
    exp_b = (sb0, sb1, sb2, sb3)

    _stage(
        [
            (ei3.at[0].at[pl.ds(rowbase, RF2)], src2_v),
            (ei3.at[1].at[pl.ds(rowbase, RF2)], dst2_v),
            (h0p.at[chunk], h_c),
            (wb, wb_v),
        ],
        sem,
    )
    _fill(ones_v, SCHUNK // 16, 1.0)
    _fill(zero_v, CN // 16, 0.0)

    @pl.when(sid == 0)
    def _():
        @pl.when(cid == 0)
        def _():
            pltpu.sync_copy(zero_v.at[pl.ds(0, 16)], fA)

        @pl.when(cid == 1)
        def _():
            pltpu.sync_copy(zero_v.at[pl.ds(0, 16)], fB)

    pltpu.sync_copy(zero_v, agg_sp.at[chunk])
    plsc.subcore_barrier()

    def exchange(r):
        @pl.when(cid == 0)
        def _():
            pltpu.sync_copy(s_c, exp_a[r].at[chunk])

        @pl.when(cid == 1)
        def _():
            pltpu.sync_copy(s_c, exp_b[r].at[chunk])

        plsc.subcore_barrier()
        lim = jnp.full((16,), float(r + 1), _f32)

        @pl.when(sid == 0)
        def _():
            flag_v[...] = lim

            @pl.when(cid == 0)
            def _():
                pltpu.sync_copy(flag_v, fA)

                def body(c):
                    pltpu.sync_copy(fB, flag_v)
                    return jnp.logical_not(jnp.all(flag_v[...] >= lim))

                lax.while_loop(lambda c: c, body, jnp.bool_(True))

            @pl.when(cid == 1)
            def _():
                pltpu.sync_copy(flag_v, fB)

                def body(c):
                    pltpu.sync_copy(fA, flag_v)
                    return jnp.logical_not(jnp.all(flag_v[...] >= lim))

                lax.while_loop(lambda c: c, body, jnp.bool_(True))

        plsc.subcore_barrier()

        @pl.when(cid == 0)
        def _():
            pltpu.sync_copy(exp_b[r].at[chunk], o_c)

        @pl.when(cid == 1)
        def _():
            pltpu.sync_copy(exp_a[r].at[chunk], o_c)

    _scatter_ones_rows(ones_v, dst2_v, agg_sp, sem, 0, RF2, 10)
    plsc.subcore_barrier()
    pltpu.sync_copy(agg_sp.at[chunk], s_c)
    pltpu.sync_copy(zero_v, agg_sp.at[chunk])
    exchange(0)

    @plsc.parallel_loop(0, CN // 16, unroll=4)
    def _(i):
        o = pl.ds(i * 16, 16)
        dd = _rsqrt16(s_c[o] + o_c[o] + 1.0)
        d_c[o] = dd
        s_c[o] = dd * h_c[o]

    pltpu.sync_copy(s_c, u0.at[chunk])
    plsc.subcore_barrier()

    for r in (1, 2, 3):
        pltpu.sync_copy(u_bufs[r - 1], u_v)
        G2 = 8

        def gs(g, _):
            base = g * G2

            @plsc.parallel_loop(base, base + G2, unroll=2)
            def _(rr):
                for c in range(8):
                    o = pl.ds(c * 16, 16)
                    msg2_v[rr, o] = plsc.load_gather(u_v, [src2_v[rr, o]])

            for j in range(G2):
                pltpu.async_copy(
                    msg2_v.at[base + j],
                    agg_sp.at[dst2_v.at[base + j]],
                    sem,
                    add=True,
                )
            return 0

        lax.fori_loop(0, RF2 // G2, gs, 0)

        def drain(g, _):
            base = g * G2
            for j in range(G2):
                pltpu.make_async_copy(
                    msg2_v.at[base + j],
                    agg_sp.at[dst2_v.at[base + j]],
                    sem,
                ).wait()
            return 0

        lax.fori_loop(0, RF2 // G2, drain, 0)
        plsc.subcore_barrier()
        pltpu.sync_copy(agg_sp.at[chunk], s_c)
        if r < 3:
            pltpu.sync_copy(zero_v, agg_sp.at[chunk])
        exchange(r)

        if r < 3:
            wv = wb_v[pl.ds((r - 1) * 32, 16)]
            bv = wb_v[pl.ds((r - 1) * 32 + 16, 16)]

            @plsc.parallel_loop(0, CN // 16, unroll=4)
            def _(i):
                o = pl.ds(i * 16, 16)
                dd = d_c[o]
                hp = h_c[o]
                s = s_c[o] + o_c[o]
                z = (1.0 - ALPHA) * (dd * s + dd * dd * hp) + ALPHA * hp
                hr = z * wv + bv
                h_c[o] = hr
                s_c[o] = dd * hr

            pltpu.sync_copy(s_c, u_bufs[r].at[chunk])
            plsc.subcore_barrier()
        else:

            @plsc.parallel_loop(0, CN // 16, unroll=4)
            def _(i):
                o = pl.ds(i * 16, 16)
                dd = d_c[o]
                hp = h_c[o]
                s = s_c[o] + o_c[o]
                z = (1.0 - ALPHA) * (dd * s + dd * dd * hp) + ALPHA * hp
                s_c[o] = jnp.maximum(z, 0.0) + 0.001

            @pl.when((cid == 0) & (sid < 15))
            def _():
                pltpu.sync_copy(s_c, y.at[pl.ds(sid * CN, CN)])

            @pl.when((cid == 0) & (sid == 15))
            def _():
                pltpu.sync_copy(
                    s_c.at[pl.ds(0, N - 15 * CN)],
                    y.at[pl.ds(15 * CN, N - 15 * CN)],
                )


EW1 = EPAD // NS
NK1 = EW1 // SCHUNK

_mesh1 = plsc.VectorSubcoreMesh(
    core_axis_name="c", subcore_axis_name="s", num_cores=1
)


def _scatter_grouped(msg_v, dst2_v, agg_sp, sem, nk):
    G = 8

    def group(g, _):
        base = g * G
        descs = [
            pltpu.async_copy(
                msg_v.at[pl.ds((base + j) * SCHUNK, SCHUNK)],
                agg_sp.at[dst2_v.at[base + j]],
                sem,
                add=True,
            )
            for j in range(G)
        ]
        for dsc in descs:
            dsc.wait()
        return 0

    lax.fori_loop(0, nk // G, group, 0)


def _scatter_ones_grouped(ones_v, dst2_v, agg_sp, sem, nk):
    G = 8

    def group(g, _):
        base = g * G
        descs = [
            pltpu.async_copy(
                ones_v, agg_sp.at[dst2_v.at[base + j]], sem, add=True
            )
            for j in range(G)
        ]
        for dsc in descs:
            dsc.wait()
        return 0

    lax.fori_loop(0, nk // G, group, 0)


@functools.partial(
    pl.kernel,
    out_type=(_sds((NPAD,), _f32), _sds((NPAD,), _f32)),
    mesh=_mesh1,
    compiler_params=_sc_params,
    scratch_types=[
        pltpu.VMEM((EW1,), jnp.int32),
        pltpu.VMEM((NK1, SCHUNK), jnp.int32),
        pltpu.VMEM((EW1,), _f32),
        pltpu.VMEM((NPAD,), _f32),
        pltpu.VMEM((SCHUNK,), _f32),
        pltpu.VMEM((CN,), _f32),
        pltpu.VMEM((CN,), _f32),
        pltpu.VMEM((CN,), _f32),
        pltpu.VMEM((CN,), _f32),
        pltpu.VMEM((64,), _f32),
        pltpu.VMEM_SHARED((NPAD,), _f32),
        pltpu.SemaphoreType.DMA,
    ],
)
def _mega_kernel(src2, dst3, h0p, wb, y, u_hbm,
                 src_v, dst2_v, msg_v, u_v, ones_v, d_c, h_c, s_c,
                 zero_v, wb_v, agg_sp, sem):
    sid = lax.axis_index("s")
    chunk = pl.ds(sid * CN, CN)
    _stage(
        [
            (src2.at[sid], src_v),
            (dst3.at[sid], dst2_v),
            (h0p.at[chunk], h_c),
            (wb, wb_v),
        ],
        sem,
    )
    _fill(ones_v, SCHUNK // 16, 1.0)
    _fill(zero_v, CN // 16, 0.0)
    pltpu.sync_copy(zero_v, agg_sp.at[chunk])
    plsc.subcore_barrier()

    _scatter_ones_grouped(ones_v, dst2_v, agg_sp, sem, NK1)
    plsc.subcore_barrier()
    pltpu.sync_copy(agg_sp.at[chunk], s_c)

    @plsc.parallel_loop(0, CN // 16, unroll=4)
    def _(i):
        o = pl.ds(i * 16, 16)
        dd = _rsqrt16(s_c[o] + 1.0)
        d_c[o] = dd
        s_c[o] = dd * h_c[o]

    pltpu.sync_copy(s_c, u_hbm.at[chunk])
    pltpu.sync_copy(zero_v, agg_sp.at[chunk])
    plsc.subcore_barrier()

    for r in (1, 2, 3):
        pltpu.sync_copy(u_hbm, u_v)

        @plsc.parallel_loop(0, EW1 // 16, unroll=8)
        def _(i):
            o = pl.ds(i * 16, 16)
            msg_v[o] = plsc.load_gather(u_v, [src_v[o]])

        _scatter_grouped(msg_v, dst2_v, agg_sp, sem, NK1)
        plsc.subcore_barrier()
        pltpu.sync_copy(agg_sp.at[chunk], s_c)

        if r < 3:
            wv = wb_v[pl.ds((r - 1) * 32, 16)]
            bv = wb_v[pl.ds((r - 1) * 32 + 16, 16)]

            @plsc.parallel_loop(0, CN // 16, unroll=4)
            def _(i):
                o = pl.ds(i * 16, 16)
                dd = d_c[o]
                hp = h_c[o]
                z = (1.0 - ALPHA) * (dd * s_c[o] + dd * dd * hp) + ALPHA * hp
                hr = z * wv + bv
                h_c[o] = hr
                s_c[o] = dd * hr

            pltpu.sync_copy(s_c, u_hbm.at[chunk])
            pltpu.sync_copy(zero_v, agg_sp.at[chunk])
            plsc.subcore_barrier()
        else:

            @plsc.parallel_loop(0, CN // 16, unroll=4)
            def _(i):
                o = pl.ds(i * 16, 16)
                dd = d_c[o]
                hp = h_c[o]
                z = (1.0 - ALPHA) * (dd * s_c[o] + dd * dd * hp) + ALPHA * hp
                s_c[o] = jnp.maximum(z, 0.0) + 0.001

            pltpu.sync_copy(s_c, y.at[chunk])


def _wb_vec(W, b):
    return jnp.concatenate(
        [
            jnp.broadcast_to(W.reshape(-1)[:1], (16,)),
            jnp.broadcast_to(b.reshape(-1)[:1], (16,)),
        ]
    ).astype(_f32)


def kernel(x, edge_index, W0, b0, W1, b1, W2, b2):
    padc = (jnp.arange((ERP - ER) * SCHUNK, dtype=jnp.int32) % (NPAD - N) + N
            ).reshape(1, ERP - ER, SCHUNK)
    ei3 = jnp.concatenate(
        [
            edge_index.astype(jnp.int32).reshape(2, ER, SCHUNK),
            jnp.broadcast_to(padc, (2, ERP - ER, SCHUNK)),
        ],
        axis=1,
    )
    h0p = _h0_tc(x, W0.reshape(1, D), b0)
    wb = jnp.concatenate([_wb_vec(W1, b1), _wb_vec(W2, b2)])
    outs = _mega3_kernel(ei3, h0p, wb)
    return outs[0].reshape(N, 1)
